# Initial kernel scaffold; baseline (speedup 1.0000x reference)
import functools

import jax
import jax.numpy as jnp
from jax import lax
from jax.experimental import pallas as pl
from jax.experimental.pallas import tpu as pltpu
from jax.experimental.pallas import tpu_sc as plsc

_D = 32
_G = 128
_K = 10
_R = _G * _K


@functools.partial(jax.jit, static_argnums=(2, 3))
def _sc_gather(xg, embeds, n_rows, n_workers):
    groups = xg.shape[0]
    groups_per_worker = groups // n_workers
    chunks = groups_per_worker // _K

    mesh = plsc.VectorSubcoreMesh(core_axis_name="c", subcore_axis_name="s")

    @functools.partial(
        pl.kernel,
        out_type=jax.ShapeDtypeStruct((n_rows, _D), jnp.float32),
        mesh=mesh,
        scratch_types=[
            pltpu.VMEM((_K, _G), jnp.int32),
            pltpu.VMEM((_R, _D), jnp.float32),
            pltpu.SemaphoreType.DMA,
        ],
    )
    def body(x_hbm, tab_hbm, out_hbm, idx_v, rows_v, gsem):
        wid = lax.axis_index("s") * mesh.num_cores + lax.axis_index("c")
        grp_base = wid * groups_per_worker

        def chunk(c, carry):
            grp0 = grp_base + c * _K
            pltpu.sync_copy(x_hbm.at[pl.ds(grp0, _K)], idx_v)
            copies = [
                pltpu.async_copy(
                    tab_hbm.at[idx_v.at[g]],
                    rows_v.at[pl.ds(g * _G, _G)],
                    gsem,
                )
                for g in range(_K)
            ]
            for cp in copies:
                cp.wait()
            pltpu.sync_copy(rows_v, out_hbm.at[pl.ds(grp0 * _G, _R)])
            return carry

        lax.fori_loop(0, chunks, chunk, 0)

    return body(xg, embeds)


def kernel(x, embeds):
    n = x.size
    xg = x.reshape(-1).astype(jnp.int32).reshape(n // _G, _G)
    return _sc_gather(xg, embeds, n, 32)


# speedup vs baseline: 1.6449x; 1.6449x over previous
import functools

import jax
import jax.numpy as jnp
from jax import lax
from jax.experimental import pallas as pl
from jax.experimental.pallas import tpu as pltpu
from jax.experimental.pallas import tpu_sc as plsc

_D = 32
_G = 128
_K = 8
_R = _G * _K


@functools.partial(jax.jit, static_argnums=(2, 3))
def _sc_gather(xg, embeds, n_rows, n_workers):
    groups = xg.shape[0]
    groups_per_worker = groups // n_workers
    chunks = groups_per_worker // _K

    mesh = plsc.VectorSubcoreMesh(core_axis_name="c", subcore_axis_name="s")

    @functools.partial(
        pl.kernel,
        out_type=jax.ShapeDtypeStruct((n_rows, _D), jnp.float32),
        mesh=mesh,
        scratch_types=[
            pltpu.VMEM((_K, _G), jnp.int32),
            pltpu.VMEM((_R, _D), jnp.float32),
            pltpu.SemaphoreType.DMA,
        ],
        compiler_params=pltpu.CompilerParams(use_tc_tiling_on_sc=False),
    )
    def body(x_hbm, tab_hbm, out_hbm, idx_v, rows_v, gsem):
        wid = lax.axis_index("s") * mesh.num_cores + lax.axis_index("c")
        grp_base = wid * groups_per_worker

        def chunk(c, carry):
            grp0 = grp_base + c * _K
            pltpu.sync_copy(x_hbm.at[pl.ds(grp0, _K)], idx_v)
            copies = [
                pltpu.async_copy(
                    tab_hbm.at[idx_v.at[g]],
                    rows_v.at[pl.ds(g * _G, _G)],
                    gsem,
                )
                for g in range(_K)
            ]
            for cp in copies:
                cp.wait()
            pltpu.sync_copy(rows_v, out_hbm.at[pl.ds(grp0 * _G, _R)])
            return carry

        lax.fori_loop(0, chunks, chunk, 0)

    return body(xg, embeds)


def kernel(x, embeds):
    n = x.size
    xg = x.reshape(-1).astype(jnp.int32).reshape(n // _G, _G)
    return _sc_gather(xg, embeds, n, 32)


# speedup vs baseline: 1.7215x; 1.0466x over previous
import functools

import jax
import jax.numpy as jnp
from jax import lax
from jax.experimental import pallas as pl
from jax.experimental.pallas import tpu as pltpu
from jax.experimental.pallas import tpu_sc as plsc

_D = 32
_G = 128
_K = 8
_R = _G * _K


@functools.partial(jax.jit, static_argnums=(2, 3))
def _sc_gather(xg, embeds, n_rows, n_workers):
    groups = xg.shape[0]
    groups_per_worker = groups // n_workers
    chunks = groups_per_worker // _K

    mesh = plsc.VectorSubcoreMesh(core_axis_name="c", subcore_axis_name="s")

    @functools.partial(
        pl.kernel,
        out_type=jax.ShapeDtypeStruct((n_rows, _D), jnp.float32),
        mesh=mesh,
        scratch_types=[
            pltpu.VMEM((2, _K, _G), jnp.int32),
            pltpu.VMEM((2, _R, _D), jnp.float32),
            pltpu.SemaphoreType.DMA,
            pltpu.SemaphoreType.DMA,
            pltpu.SemaphoreType.DMA,
            pltpu.SemaphoreType.DMA,
            pltpu.SemaphoreType.DMA,
        ],
        compiler_params=pltpu.CompilerParams(use_tc_tiling_on_sc=False),
    )
    def body(x_hbm, tab_hbm, out_hbm, idx_v, rows_v, gsem, i0, i1, o0, o1):
        wid = lax.axis_index("s") * mesh.num_cores + lax.axis_index("c")
        grp_base = wid * groups_per_worker
        isems = (i0, i1)
        osems = (o0, o1)

        def icopy(c, b):
            grp0 = grp_base + c * _K
            return pltpu.make_async_copy(
                x_hbm.at[pl.ds(grp0, _K)], idx_v.at[b], isems[b]
            )

        def ocopy(c, b):
            grp0 = grp_base + c * _K
            return pltpu.make_async_copy(
                rows_v.at[b], out_hbm.at[pl.ds(grp0 * _G, _R)], osems[b]
            )

        def process(c, b, wait_out):
            icopy(c, b).wait()
            if wait_out:
                ocopy(c, b).wait()
            copies = [
                pltpu.async_copy(
                    tab_hbm.at[idx_v.at[b].at[g]],
                    rows_v.at[b].at[pl.ds(g * _G, _G)],
                    gsem,
                )
                for g in range(_K)
            ]
            for cp in copies:
                cp.wait()
            ocopy(c, b).start()
            nxt = jnp.minimum(c + 2, chunks - 1)
            icopy(nxt, b).start()

        icopy(0, 0).start()
        icopy(1, 1).start()
        process(0, 0, False)
        process(1, 1, False)

        def step(i, carry):
            process(2 * i, 0, True)
            process(2 * i + 1, 1, True)
            return carry

        lax.fori_loop(1, chunks // 2, step, 0)
        icopy(chunks - 1, 0).wait()
        icopy(chunks - 1, 1).wait()
        ocopy(chunks - 2, 0).wait()
        ocopy(chunks - 1, 1).wait()

    return body(xg, embeds)


def kernel(x, embeds):
    n = x.size
    xg = x.reshape(-1).astype(jnp.int32).reshape(n // _G, _G)
    return _sc_gather(xg, embeds, n, 32)


# speedup vs baseline: 1.7219x; 1.0002x over previous
import functools

import jax
import jax.numpy as jnp
from jax import lax
from jax.experimental import pallas as pl
from jax.experimental.pallas import tpu as pltpu
from jax.experimental.pallas import tpu_sc as plsc

_D = 32
_R = 1024


@functools.partial(jax.jit, static_argnums=(2, 3))
def _sc_gather(xf, embeds, n_rows, n_workers):
    rows_per_worker = n_rows // n_workers
    chunks = rows_per_worker // _R

    mesh = plsc.VectorSubcoreMesh(core_axis_name="c", subcore_axis_name="s")

    @functools.partial(
        pl.kernel,
        out_type=jax.ShapeDtypeStruct((n_rows, _D), jnp.float32),
        mesh=mesh,
        scratch_types=[
            pltpu.VMEM((2, _R), jnp.int32),
            pltpu.VMEM((2, _R, _D), jnp.float32),
            pltpu.SemaphoreType.DMA,
            pltpu.SemaphoreType.DMA,
            pltpu.SemaphoreType.DMA,
            pltpu.SemaphoreType.DMA,
            pltpu.SemaphoreType.DMA,
        ],
        compiler_params=pltpu.CompilerParams(use_tc_tiling_on_sc=False),
    )
    def body(x_hbm, tab_hbm, out_hbm, idx_v, rows_v, gsem, i0, i1, o0, o1):
        wid = lax.axis_index("s") * mesh.num_cores + lax.axis_index("c")
        row_base = wid * rows_per_worker
        isems = (i0, i1)
        osems = (o0, o1)

        def icopy(c, b):
            row0 = row_base + c * _R
            return pltpu.make_async_copy(
                x_hbm.at[pl.ds(row0, _R)], idx_v.at[b], isems[b]
            )

        def ocopy(c, b):
            row0 = row_base + c * _R
            return pltpu.make_async_copy(
                rows_v.at[b], out_hbm.at[pl.ds(row0, _R)], osems[b]
            )

        def process(c, b, wait_out):
            icopy(c, b).wait()
            if wait_out:
                ocopy(c, b).wait()
            pltpu.async_copy(tab_hbm.at[idx_v.at[b]], rows_v.at[b], gsem).wait()
            ocopy(c, b).start()
            nxt = jnp.minimum(c + 2, chunks - 1)
            icopy(nxt, b).start()

        icopy(0, 0).start()
        icopy(1, 1).start()
        process(0, 0, False)
        process(1, 1, False)

        def step(i, carry):
            process(2 * i, 0, True)
            process(2 * i + 1, 1, True)
            return carry

        lax.fori_loop(1, chunks // 2, step, 0)
        icopy(chunks - 1, 0).wait()
        icopy(chunks - 1, 1).wait()
        ocopy(chunks - 2, 0).wait()
        ocopy(chunks - 1, 1).wait()

    return body(xf, embeds)


def kernel(x, embeds):
    n = x.size
    xf = x.reshape(-1).astype(jnp.int32)
    return _sc_gather(xf, embeds, n, 32)
